# baseline (speedup 1.0000x reference)
import jax
import jax.numpy as jnp
from jax.experimental import pallas as pl
from jax.experimental.pallas import tpu as pltpu


_EB = 512


def _body(x_ref, t_ref, o_ref, tt_ref):
    bi = pl.program_id(1)

    @pl.when(bi == 0)
    def _():
        tt_ref[...] = t_ref[...].T

    o_ref[...] = x_ref[...] + tt_ref[...]


def kernel(x, table):
    b, e, l = x.shape
    x2 = x.reshape(b * e, l)
    ne = e // _EB
    out = pl.pallas_call(
        _body,
        grid=(ne, b),
        in_specs=[
            pl.BlockSpec((_EB, l), lambda ei, bi: (bi * ne + ei, 0)),
            pl.BlockSpec((l, _EB), lambda ei, bi: (0, ei)),
        ],
        out_specs=pl.BlockSpec((_EB, l), lambda ei, bi: (bi * ne + ei, 0)),
        out_shape=jax.ShapeDtypeStruct((b * e, l), x.dtype),
        scratch_shapes=[pltpu.VMEM((_EB, l), jnp.float32)],
    )(x2, table)
    return out.reshape(b, e, l)


# speedup vs baseline: 1.0730x; 1.0730x over previous
import jax
import jax.numpy as jnp
from jax.experimental import pallas as pl


_EB = 128
_LB = 4096


def _body(x_ref, t_ref, o_ref):
    t = t_ref[...]
    o_ref[...] = x_ref[...] + t.T[None, :, :]


def kernel(x, table):
    b, e, l = x.shape
    grid = (e // _EB, l // _LB)
    return pl.pallas_call(
        _body,
        grid=grid,
        in_specs=[
            pl.BlockSpec((b, _EB, _LB), lambda ei, li: (0, ei, li)),
            pl.BlockSpec((_LB, _EB), lambda ei, li: (li, ei)),
        ],
        out_specs=pl.BlockSpec((b, _EB, _LB), lambda ei, li: (0, ei, li)),
        out_shape=jax.ShapeDtypeStruct(x.shape, x.dtype),
    )(x, table)
